# baseline (speedup 1.0000x reference)
import functools

import jax
import jax.numpy as jnp
from jax import lax
from jax.experimental import pallas as pl
from jax.experimental.pallas import tpu as pltpu
from jax.experimental.pallas import tpu_sc as plsc

_NC = 2
_NS = 16
_LANES = 16
_CHUNK = 80
_GRP = 10
_HW = 128
_W = _NC * _HW


def _node_mm(h, wa_stack, n_block):
    n, nd = h.shape
    nb = n // n_block

    def body(h_ref, w_ref, o_ref):
        o_ref[...] = jnp.dot(h_ref[...], w_ref[0],
                             preferred_element_type=jnp.float32)

    return pl.pallas_call(
        body,
        grid=(_NC, nb),
        in_specs=[
            pl.BlockSpec((n_block, nd), lambda c, i: (i, 0)),
            pl.BlockSpec((1, nd, _HW), lambda c, i: (c, 0, 0)),
        ],
        out_specs=pl.BlockSpec((n_block, _HW), lambda c, i: (c * nb + i, 0)),
        out_shape=jax.ShapeDtypeStruct((_NC * n, _HW), jnp.float32),
    )(h, wa_stack)


def _edge_mm(e, wb_stack, br_stack, e_block):
    ne, ed = e.shape

    def body(e_ref, w_ref, b_ref, q_ref, eo_ref):
        eb = e_ref[...]
        q_ref[0] = jnp.dot(eb, w_ref[0],
                           preferred_element_type=jnp.float32) + b_ref[0]
        eo_ref[...] = jnp.maximum(eb, 0.0)

    return pl.pallas_call(
        body,
        grid=(_NC, ne // e_block),
        in_specs=[
            pl.BlockSpec((e_block, ed), lambda c, i: (i, 0)),
            pl.BlockSpec((1, ed, _HW), lambda c, i: (c, 0, 0)),
            pl.BlockSpec((1, 1, _HW), lambda c, i: (c, 0, 0)),
        ],
        out_specs=[
            pl.BlockSpec((1, e_block, _HW), lambda c, i: (c, i, 0)),
            pl.BlockSpec((e_block, ed), lambda c, i: (i, 0)),
        ],
        out_shape=[
            jax.ShapeDtypeStruct((_NC, ne, _HW), jnp.float32),
            jax.ShapeDtypeStruct((ne, ed), jnp.float32),
        ],
    )(e, wb_stack, br_stack)


def _out_mm(s_stack, w2top, w2bot, n_block):
    n = s_stack.shape[0] // _NC
    nd = w2top.shape[1]
    nb = n // n_block

    def body(a_ref, b_ref, wt_ref, wb_ref, o_ref):
        acc = jnp.dot(a_ref[...], wt_ref[...],
                      preferred_element_type=jnp.float32)
        acc = acc + jnp.dot(b_ref[...], wb_ref[...],
                            preferred_element_type=jnp.float32)
        o_ref[...] = jnp.maximum(acc, 0.0)

    return pl.pallas_call(
        body,
        grid=(nb,),
        in_specs=[
            pl.BlockSpec((n_block, _HW), lambda i: (i, 0)),
            pl.BlockSpec((n_block, _HW), lambda i: (nb + i, 0)),
            pl.BlockSpec((_HW, nd), lambda i: (0, 0)),
            pl.BlockSpec((_HW, nd), lambda i: (0, 0)),
        ],
        out_specs=pl.BlockSpec((n_block, nd), lambda i: (i, 0)),
        out_shape=jax.ShapeDtypeStruct((n, nd), jnp.float32),
    )(s_stack, s_stack, w2top, w2bot)


def _sc_segment_sum(src1d, dst1d, p, q, n, nchunks):
    nk = nchunks // _NS
    assert nk % _GRP == 0
    ngroups = nk // _GRP
    rz = (n // _NS) // 8 * 8
    nextra = (n - rz * _NS) // 8
    ncol = _HW // _LANES
    nci = _CHUNK // _LANES

    mesh = plsc.VectorSubcoreMesh(core_axis_name="c", subcore_axis_name="s")

    @functools.partial(
        pl.kernel,
        mesh=mesh,
        out_type=jax.ShapeDtypeStruct((_NC * n, _HW), jnp.float32),
        scratch_types=[
            pltpu.VMEM((_GRP * _CHUNK,), jnp.int32),
            pltpu.VMEM((_GRP * _CHUNK,), jnp.int32),
            pltpu.VMEM((_CHUNK,), jnp.int32),
            pltpu.VMEM((_CHUNK,), jnp.int32),
            pltpu.VMEM((_CHUNK,), jnp.int32),
            pltpu.VMEM((_CHUNK,), jnp.int32),
            pltpu.VMEM((_CHUNK, _HW), jnp.float32),
            pltpu.VMEM((_CHUNK, _HW), jnp.float32),
            pltpu.VMEM((_CHUNK, _HW), jnp.float32),
            pltpu.VMEM((_CHUNK, _HW), jnp.float32),
            pltpu.VMEM_SHARED((n, _HW), jnp.float32),
            pltpu.SemaphoreType.DMA,
            pltpu.SemaphoreType.DMA,
            pltpu.SemaphoreType.DMA,
            pltpu.SemaphoreType.DMA,
            pltpu.SemaphoreType.DMA,
            pltpu.SemaphoreType.DMA,
        ],
        compiler_params=pltpu.CompilerParams(use_tc_tiling_on_sc=False),
    )
    def sc_kernel(src_hbm, dst_hbm, p_hbm, q_hbm, out_hbm,
                  sbig, dbig, ci0, ci1, di0, di1, pr0, pr1, qr0, qr1, acc,
                  sg0, sg1, sq0, sq1, ss0, ss1):
        c = lax.axis_index("c")
        s = lax.axis_index("s")
        cidx = (ci0, ci1)
        didx = (di0, di1)
        prows = (pr0, pr1)
        qrows = (qr0, qr1)
        sg = (sg0, sg1)
        sq = (sq0, sq1)
        ss = (ss0, ss1)
        cn = c * n
        ibase = s * (nk * _CHUNK)
        qbase = s * nk

        def zrow(j, carry):
            for cc in range(ncol):
                qr0[j, pl.ds(cc * _LANES, _LANES)] = jnp.zeros(
                    (_LANES,), jnp.float32)
            return carry
        lax.fori_loop(0, _CHUNK, zrow, 0)

        r0 = s * rz
        rem = rz % _CHUNK
        nfull = rz // _CHUNK
        for k in range(nfull):
            pltpu.sync_copy(qr0, acc.at[pl.ds(r0 + k * _CHUNK, _CHUNK)])
        if rem:
            pltpu.sync_copy(qr0.at[pl.ds(0, rem)],
                            acc.at[pl.ds(r0 + nfull * _CHUNK, rem)])

        @pl.when(s < nextra)
        def _():
            pltpu.sync_copy(qr0.at[pl.ds(0, 8)],
                            acc.at[pl.ds(rz * _NS + s * 8, 8)])
        plsc.subcore_barrier()

        def build_idx(j, b):
            for cc in range(nci):
                sl = pl.ds(cc * _LANES, _LANES)
                gsl = pl.ds(j * _CHUNK + cc * _LANES, _LANES)
                cidx[b][sl] = sbig[gsl] + cn
                didx[b][sl] = dbig[gsl]

        def issue(k, b):
            pltpu.async_copy(p_hbm.at[cidx[b]], prows[b], sg[b])
            pltpu.async_copy(
                q_hbm.at[c, pl.ds((qbase + k) * _CHUNK, _CHUNK)],
                qrows[b], sq[b])

        def wait_scatter(b):
            pltpu.make_async_copy(prows[b], acc.at[didx[b]], ss[b]).wait()

        def group(g, carry):
            g0 = g * _GRP
            pltpu.sync_copy(
                src_hbm.at[pl.ds(ibase + g0 * _CHUNK, _GRP * _CHUNK)], sbig)
            pltpu.sync_copy(
                dst_hbm.at[pl.ds(ibase + g0 * _CHUNK, _GRP * _CHUNK)], dbig)
            for j in range(_GRP):
                k = g0 + j
                b = j % 2

                if j == 0:
                    @pl.when(k >= 2)
                    def _():
                        wait_scatter(b)
                    build_idx(0, b)
                    issue(k, b)

                pltpu.make_async_copy(
                    p_hbm.at[cidx[b]], prows[b], sg[b]).wait()
                pltpu.make_async_copy(
                    q_hbm.at[c, pl.ds((qbase + k) * _CHUNK, _CHUNK)],
                    qrows[b], sq[b]).wait()

                def row(jj, rc):
                    for cc in range(ncol):
                        sl = pl.ds(cc * _LANES, _LANES)
                        prows[b][jj, sl] = jnp.maximum(
                            prows[b][jj, sl] + qrows[b][jj, sl], 0.0)
                    return rc
                lax.fori_loop(0, _CHUNK, row, 0, unroll=2)

                pltpu.async_copy(prows[b], acc.at[didx[b]], ss[b], add=True)

                if j < _GRP - 1:
                    @pl.when(k >= 1)
                    def _():
                        wait_scatter(1 - b)
                    build_idx(j + 1, 1 - b)
                    issue(k + 1, 1 - b)
            return carry
        lax.fori_loop(0, ngroups, group, 0)

        wait_scatter(0)
        wait_scatter(1)
        plsc.subcore_barrier()

        ob = cn + r0
        for k in range(nfull):
            pltpu.sync_copy(acc.at[pl.ds(r0 + k * _CHUNK, _CHUNK)],
                            out_hbm.at[pl.ds(ob + k * _CHUNK, _CHUNK)])
        if rem:
            pltpu.sync_copy(acc.at[pl.ds(r0 + nfull * _CHUNK, rem)],
                            out_hbm.at[pl.ds(ob + nfull * _CHUNK, rem)])

        @pl.when(s < nextra)
        def _():
            pltpu.sync_copy(acc.at[pl.ds(rz * _NS + s * 8, 8)],
                            out_hbm.at[pl.ds(cn + rz * _NS + s * 8, 8)])

    return sc_kernel(src1d, dst1d, p, q)


def kernel(h, edge_index, e, W1, b1, gamma, beta, run_mean, run_var, W2, b2):
    n, nd = h.shape
    ne, ed = e.shape
    emb = W1.shape[0]

    scale = gamma * lax.rsqrt(run_var + 1e-5)
    shift = beta - run_mean * scale
    w1s = W1 * scale[:, None]
    beff = b1 * scale + shift

    wa = jnp.zeros((nd, _W), jnp.float32).at[:, :emb].set(w1s[:, :nd].T)
    wb = jnp.zeros((ed, _W), jnp.float32).at[:, :emb].set(w1s[:, nd:].T)
    brow = jnp.zeros((1, _W), jnp.float32).at[0, :emb].set(beff)
    brow = brow.at[0, emb].set(1.0)
    w2e = jnp.zeros((_W, nd), jnp.float32).at[:emb, :].set(W2.T)
    w2e = w2e.at[emb, :].set(b2)
    wa_stack = jnp.stack([wa[:, :_HW], wa[:, _HW:]])
    wb_stack = jnp.stack([wb[:, :_HW], wb[:, _HW:]])
    br_stack = jnp.stack([brow[:, :_HW], brow[:, _HW:]])

    nchunks = ne // _CHUNK
    src1d = edge_index[0].astype(jnp.int32)
    dst1d = edge_index[1].astype(jnp.int32)

    p = _node_mm(h, wa_stack, n_block=1000)
    q, e_out = _edge_mm(e, wb_stack, br_stack, e_block=2000)
    s_stack = _sc_segment_sum(src1d, dst1d, p, q, n, nchunks)
    h_out = _out_mm(s_stack, w2e[:_HW], w2e[_HW:], n_block=1000)
    return (h_out, e_out)


# speedup vs baseline: 1.1414x; 1.1414x over previous
import functools

import jax
import jax.numpy as jnp
from jax import lax
from jax.experimental import pallas as pl
from jax.experimental.pallas import tpu as pltpu
from jax.experimental.pallas import tpu_sc as plsc

_NC = 2
_NS = 16
_LANES = 16
_CHUNK = 80
_GRP = 10
_HW = 128
_W = _NC * _HW


def _node_mm(h, wa_stack, n_block):
    n, nd = h.shape
    nb = n // n_block

    def body(h_ref, w_ref, o_ref):
        o_ref[...] = jnp.dot(h_ref[...], w_ref[0],
                             preferred_element_type=jnp.float32)

    return pl.pallas_call(
        body,
        grid=(_NC, nb),
        in_specs=[
            pl.BlockSpec((n_block, nd), lambda c, i: (i, 0)),
            pl.BlockSpec((1, nd, _HW), lambda c, i: (c, 0, 0)),
        ],
        out_specs=pl.BlockSpec((n_block, _HW), lambda c, i: (c * nb + i, 0)),
        out_shape=jax.ShapeDtypeStruct((_NC * n, _HW), jnp.float32),
    )(h, wa_stack)


def _edge_mm(e, wb_stack, br_stack, e_block):
    ne, ed = e.shape

    def body(e_ref, w_ref, b_ref, q_ref, eo_ref):
        eb = e_ref[...]
        q_ref[0] = jnp.dot(eb, w_ref[0],
                           preferred_element_type=jnp.float32) + b_ref[0]
        eo_ref[...] = jnp.maximum(eb, 0.0)

    return pl.pallas_call(
        body,
        grid=(_NC, ne // e_block),
        in_specs=[
            pl.BlockSpec((e_block, ed), lambda c, i: (i, 0)),
            pl.BlockSpec((1, ed, _HW), lambda c, i: (c, 0, 0)),
            pl.BlockSpec((1, 1, _HW), lambda c, i: (c, 0, 0)),
        ],
        out_specs=[
            pl.BlockSpec((1, e_block, _HW), lambda c, i: (c, i, 0)),
            pl.BlockSpec((e_block, ed), lambda c, i: (i, 0)),
        ],
        out_shape=[
            jax.ShapeDtypeStruct((_NC, ne, _HW), jnp.float32),
            jax.ShapeDtypeStruct((ne, ed), jnp.float32),
        ],
    )(e, wb_stack, br_stack)


def _out_mm(s_stack, w2top, w2bot, n_block):
    n = s_stack.shape[0] // _NC
    nd = w2top.shape[1]
    nb = n // n_block

    def body(a_ref, b_ref, wt_ref, wb_ref, o_ref):
        acc = jnp.dot(a_ref[...], wt_ref[...],
                      preferred_element_type=jnp.float32)
        acc = acc + jnp.dot(b_ref[...], wb_ref[...],
                            preferred_element_type=jnp.float32)
        o_ref[...] = jnp.maximum(acc, 0.0)

    return pl.pallas_call(
        body,
        grid=(nb,),
        in_specs=[
            pl.BlockSpec((n_block, _HW), lambda i: (i, 0)),
            pl.BlockSpec((n_block, _HW), lambda i: (nb + i, 0)),
            pl.BlockSpec((_HW, nd), lambda i: (0, 0)),
            pl.BlockSpec((_HW, nd), lambda i: (0, 0)),
        ],
        out_specs=pl.BlockSpec((n_block, nd), lambda i: (i, 0)),
        out_shape=jax.ShapeDtypeStruct((n, nd), jnp.float32),
    )(s_stack, s_stack, w2top, w2bot)


def _sc_segment_sum(src1d, dst1d, p, q, n, nchunks):
    nk = nchunks // _NS
    assert nk % _GRP == 0
    ngroups = nk // _GRP
    rz = (n // _NS) // 8 * 8
    nextra = (n - rz * _NS) // 8
    ncol = _HW // _LANES
    nci = _CHUNK // _LANES

    mesh = plsc.VectorSubcoreMesh(core_axis_name="c", subcore_axis_name="s")

    @functools.partial(
        pl.kernel,
        mesh=mesh,
        out_type=jax.ShapeDtypeStruct((_NC * n, _HW), jnp.float32),
        scratch_types=[
            pltpu.VMEM((_GRP * _CHUNK,), jnp.int32),
            pltpu.VMEM((_GRP * _CHUNK,), jnp.int32),
            pltpu.VMEM((_CHUNK,), jnp.int32),
            pltpu.VMEM((_CHUNK,), jnp.int32),
            pltpu.VMEM((_CHUNK,), jnp.int32),
            pltpu.VMEM((_CHUNK,), jnp.int32),
            pltpu.VMEM((_CHUNK, _HW), jnp.float32),
            pltpu.VMEM((_CHUNK, _HW), jnp.float32),
            pltpu.VMEM((_CHUNK, _HW), jnp.float32),
            pltpu.VMEM((_CHUNK, _HW), jnp.float32),
            pltpu.VMEM_SHARED((n, _HW), jnp.float32),
            pltpu.SemaphoreType.DMA,
            pltpu.SemaphoreType.DMA,
            pltpu.SemaphoreType.DMA,
            pltpu.SemaphoreType.DMA,
            pltpu.SemaphoreType.DMA,
            pltpu.SemaphoreType.DMA,
        ],
        compiler_params=pltpu.CompilerParams(use_tc_tiling_on_sc=False),
    )
    def sc_kernel(src_hbm, dst_hbm, p_hbm, q_hbm, out_hbm,
                  sbig, dbig, ci0, ci1, di0, di1, pr0, pr1, qr0, qr1, acc,
                  sg0, sg1, sq0, sq1, ss0, ss1):
        c = lax.axis_index("c")
        s = lax.axis_index("s")
        cidx = (ci0, ci1)
        didx = (di0, di1)
        prows = (pr0, pr1)
        qrows = (qr0, qr1)
        sg = (sg0, sg1)
        sq = (sq0, sq1)
        ss = (ss0, ss1)
        cn = c * n
        ibase = s * (nk * _CHUNK)
        qbase = s * nk

        def zrow(j, carry):
            for cc in range(ncol):
                qr0[j, pl.ds(cc * _LANES, _LANES)] = jnp.zeros(
                    (_LANES,), jnp.float32)
            return carry
        lax.fori_loop(0, _CHUNK, zrow, 0)

        r0 = s * rz
        rem = rz % _CHUNK
        nfull = rz // _CHUNK
        for k in range(nfull):
            pltpu.sync_copy(qr0, acc.at[pl.ds(r0 + k * _CHUNK, _CHUNK)])
        if rem:
            pltpu.sync_copy(qr0.at[pl.ds(0, rem)],
                            acc.at[pl.ds(r0 + nfull * _CHUNK, rem)])

        @pl.when(s < nextra)
        def _():
            pltpu.sync_copy(qr0.at[pl.ds(0, 8)],
                            acc.at[pl.ds(rz * _NS + s * 8, 8)])
        plsc.subcore_barrier()

        def build_idx(j, b):
            for cc in range(nci):
                sl = pl.ds(cc * _LANES, _LANES)
                gsl = pl.ds(j * _CHUNK + cc * _LANES, _LANES)
                cidx[b][sl] = sbig[gsl] + cn
                didx[b][sl] = dbig[gsl]

        def issue(k, b):
            pltpu.async_copy(p_hbm.at[cidx[b]], prows[b], sg[b])
            pltpu.async_copy(
                q_hbm.at[c, pl.ds((qbase + k) * _CHUNK, _CHUNK)],
                qrows[b], sq[b])

        def wait_scatter(b):
            pltpu.make_async_copy(prows[b], acc.at[didx[b]], ss[b]).wait()

        def group(g, carry):
            g0 = g * _GRP
            pltpu.sync_copy(
                src_hbm.at[pl.ds(ibase + g0 * _CHUNK, _GRP * _CHUNK)], sbig)
            pltpu.sync_copy(
                dst_hbm.at[pl.ds(ibase + g0 * _CHUNK, _GRP * _CHUNK)], dbig)
            for j in range(_GRP):
                k = g0 + j
                b = j % 2

                if j == 0:
                    @pl.when(k >= 2)
                    def _():
                        wait_scatter(b)
                    build_idx(0, b)
                    issue(k, b)

                pltpu.make_async_copy(
                    p_hbm.at[cidx[b]], prows[b], sg[b]).wait()
                pltpu.make_async_copy(
                    q_hbm.at[c, pl.ds((qbase + k) * _CHUNK, _CHUNK)],
                    qrows[b], sq[b]).wait()

                if j < _GRP - 1:
                    @pl.when(k >= 1)
                    def _():
                        wait_scatter(1 - b)
                    build_idx(j + 1, 1 - b)
                    issue(k + 1, 1 - b)

                def row(jj, rc):
                    for cc in range(ncol):
                        sl = pl.ds(cc * _LANES, _LANES)
                        prows[b][jj, sl] = jnp.maximum(
                            prows[b][jj, sl] + qrows[b][jj, sl], 0.0)
                    return rc
                lax.fori_loop(0, _CHUNK, row, 0, unroll=2)

                pltpu.async_copy(prows[b], acc.at[didx[b]], ss[b], add=True)
            return carry
        lax.fori_loop(0, ngroups, group, 0)

        wait_scatter(0)
        wait_scatter(1)
        plsc.subcore_barrier()

        ob = cn + r0
        for k in range(nfull):
            pltpu.sync_copy(acc.at[pl.ds(r0 + k * _CHUNK, _CHUNK)],
                            out_hbm.at[pl.ds(ob + k * _CHUNK, _CHUNK)])
        if rem:
            pltpu.sync_copy(acc.at[pl.ds(r0 + nfull * _CHUNK, rem)],
                            out_hbm.at[pl.ds(ob + nfull * _CHUNK, rem)])

        @pl.when(s < nextra)
        def _():
            pltpu.sync_copy(acc.at[pl.ds(rz * _NS + s * 8, 8)],
                            out_hbm.at[pl.ds(cn + rz * _NS + s * 8, 8)])

    return sc_kernel(src1d, dst1d, p, q)


def kernel(h, edge_index, e, W1, b1, gamma, beta, run_mean, run_var, W2, b2):
    n, nd = h.shape
    ne, ed = e.shape
    emb = W1.shape[0]

    scale = gamma * lax.rsqrt(run_var + 1e-5)
    shift = beta - run_mean * scale
    w1s = W1 * scale[:, None]
    beff = b1 * scale + shift

    wa = jnp.zeros((nd, _W), jnp.float32).at[:, :emb].set(w1s[:, :nd].T)
    wb = jnp.zeros((ed, _W), jnp.float32).at[:, :emb].set(w1s[:, nd:].T)
    brow = jnp.zeros((1, _W), jnp.float32).at[0, :emb].set(beff)
    brow = brow.at[0, emb].set(1.0)
    w2e = jnp.zeros((_W, nd), jnp.float32).at[:emb, :].set(W2.T)
    w2e = w2e.at[emb, :].set(b2)
    wa_stack = jnp.stack([wa[:, :_HW], wa[:, _HW:]])
    wb_stack = jnp.stack([wb[:, :_HW], wb[:, _HW:]])
    br_stack = jnp.stack([brow[:, :_HW], brow[:, _HW:]])

    nchunks = ne // _CHUNK
    src1d = edge_index[0].astype(jnp.int32)
    dst1d = edge_index[1].astype(jnp.int32)

    p = _node_mm(h, wa_stack, n_block=1000)
    q, e_out = _edge_mm(e, wb_stack, br_stack, e_block=2000)
    s_stack = _sc_segment_sum(src1d, dst1d, p, q, n, nchunks)
    h_out = _out_mm(s_stack, w2e[:_HW], w2e[_HW:], n_block=1000)
    return (h_out, e_out)


# speedup vs baseline: 1.1585x; 1.0150x over previous
import functools

import jax
import jax.numpy as jnp
from jax import lax
from jax.experimental import pallas as pl
from jax.experimental.pallas import tpu as pltpu
from jax.experimental.pallas import tpu_sc as plsc

_NC = 2
_NS = 16
_LANES = 16
_CHUNK = 80
_GRP = 25
_HW = 128
_W = _NC * _HW


def _node_mm(h, wa_stack, n_block):
    n, nd = h.shape
    nb = n // n_block

    def body(h_ref, w_ref, o_ref):
        o_ref[...] = jnp.dot(h_ref[...], w_ref[0],
                             preferred_element_type=jnp.float32)

    return pl.pallas_call(
        body,
        grid=(_NC, nb),
        in_specs=[
            pl.BlockSpec((n_block, nd), lambda c, i: (i, 0)),
            pl.BlockSpec((1, nd, _HW), lambda c, i: (c, 0, 0)),
        ],
        out_specs=pl.BlockSpec((n_block, _HW), lambda c, i: (c * nb + i, 0)),
        out_shape=jax.ShapeDtypeStruct((_NC * n, _HW), jnp.float32),
    )(h, wa_stack)


def _edge_mm(e, wb_stack, br_stack, e_block):
    ne, ed = e.shape

    def body(e_ref, w_ref, b_ref, q_ref, eo_ref):
        eb = e_ref[...]
        q_ref[0] = jnp.dot(eb, w_ref[0],
                           preferred_element_type=jnp.float32) + b_ref[0]
        eo_ref[...] = jnp.maximum(eb, 0.0)

    return pl.pallas_call(
        body,
        grid=(_NC, ne // e_block),
        in_specs=[
            pl.BlockSpec((e_block, ed), lambda c, i: (i, 0)),
            pl.BlockSpec((1, ed, _HW), lambda c, i: (c, 0, 0)),
            pl.BlockSpec((1, 1, _HW), lambda c, i: (c, 0, 0)),
        ],
        out_specs=[
            pl.BlockSpec((1, e_block, _HW), lambda c, i: (c, i, 0)),
            pl.BlockSpec((e_block, ed), lambda c, i: (i, 0)),
        ],
        out_shape=[
            jax.ShapeDtypeStruct((_NC, ne, _HW), jnp.float32),
            jax.ShapeDtypeStruct((ne, ed), jnp.float32),
        ],
    )(e, wb_stack, br_stack)


def _out_mm(s_stack, w2top, w2bot, n_block):
    n = s_stack.shape[0] // _NC
    nd = w2top.shape[1]
    nb = n // n_block

    def body(a_ref, b_ref, wt_ref, wb_ref, o_ref):
        acc = jnp.dot(a_ref[...], wt_ref[...],
                      preferred_element_type=jnp.float32)
        acc = acc + jnp.dot(b_ref[...], wb_ref[...],
                            preferred_element_type=jnp.float32)
        o_ref[...] = jnp.maximum(acc, 0.0)

    return pl.pallas_call(
        body,
        grid=(nb,),
        in_specs=[
            pl.BlockSpec((n_block, _HW), lambda i: (i, 0)),
            pl.BlockSpec((n_block, _HW), lambda i: (nb + i, 0)),
            pl.BlockSpec((_HW, nd), lambda i: (0, 0)),
            pl.BlockSpec((_HW, nd), lambda i: (0, 0)),
        ],
        out_specs=pl.BlockSpec((n_block, nd), lambda i: (i, 0)),
        out_shape=jax.ShapeDtypeStruct((n, nd), jnp.float32),
    )(s_stack, s_stack, w2top, w2bot)


def _sc_segment_sum(src1d, dst1d, p, q, n, nchunks):
    nk = nchunks // _NS
    assert nk % _GRP == 0
    ngroups = nk // _GRP
    rz = (n // _NS) // 8 * 8
    nextra = (n - rz * _NS) // 8
    ncol = _HW // _LANES
    nci = _CHUNK // _LANES

    mesh = plsc.VectorSubcoreMesh(core_axis_name="c", subcore_axis_name="s")

    @functools.partial(
        pl.kernel,
        mesh=mesh,
        out_type=jax.ShapeDtypeStruct((_NC * n, _HW), jnp.float32),
        scratch_types=[
            pltpu.VMEM((_GRP * _CHUNK,), jnp.int32),
            pltpu.VMEM((_GRP * _CHUNK,), jnp.int32),
            pltpu.VMEM((_CHUNK,), jnp.int32),
            pltpu.VMEM((_CHUNK,), jnp.int32),
            pltpu.VMEM((_CHUNK,), jnp.int32),
            pltpu.VMEM((_CHUNK,), jnp.int32),
            pltpu.VMEM((_CHUNK, _HW), jnp.float32),
            pltpu.VMEM((_CHUNK, _HW), jnp.float32),
            pltpu.VMEM((_CHUNK, _HW), jnp.float32),
            pltpu.VMEM((_CHUNK, _HW), jnp.float32),
            pltpu.VMEM_SHARED((n, _HW), jnp.float32),
            pltpu.SemaphoreType.DMA,
            pltpu.SemaphoreType.DMA,
            pltpu.SemaphoreType.DMA,
            pltpu.SemaphoreType.DMA,
            pltpu.SemaphoreType.DMA,
            pltpu.SemaphoreType.DMA,
        ],
        compiler_params=pltpu.CompilerParams(use_tc_tiling_on_sc=False),
    )
    def sc_kernel(src_hbm, dst_hbm, p_hbm, q_hbm, out_hbm,
                  sbig, dbig, ci0, ci1, di0, di1, pr0, pr1, qr0, qr1, acc,
                  sg0, sg1, sq0, sq1, ss0, ss1):
        c = lax.axis_index("c")
        s = lax.axis_index("s")
        cidx = (ci0, ci1)
        didx = (di0, di1)
        prows = (pr0, pr1)
        qrows = (qr0, qr1)
        sg = (sg0, sg1)
        sq = (sq0, sq1)
        ss = (ss0, ss1)
        cn = c * n
        ibase = s * (nk * _CHUNK)
        qbase = s * nk

        def zrow(j, carry):
            for cc in range(ncol):
                qr0[j, pl.ds(cc * _LANES, _LANES)] = jnp.zeros(
                    (_LANES,), jnp.float32)
            return carry
        lax.fori_loop(0, _CHUNK, zrow, 0)

        r0 = s * rz
        rem = rz % _CHUNK
        nfull = rz // _CHUNK
        for k in range(nfull):
            pltpu.sync_copy(qr0, acc.at[pl.ds(r0 + k * _CHUNK, _CHUNK)])
        if rem:
            pltpu.sync_copy(qr0.at[pl.ds(0, rem)],
                            acc.at[pl.ds(r0 + nfull * _CHUNK, rem)])

        @pl.when(s < nextra)
        def _():
            pltpu.sync_copy(qr0.at[pl.ds(0, 8)],
                            acc.at[pl.ds(rz * _NS + s * 8, 8)])
        plsc.subcore_barrier()

        def build_idx(j, b):
            for cc in range(nci):
                sl = pl.ds(cc * _LANES, _LANES)
                gsl = pl.ds(j * _CHUNK + cc * _LANES, _LANES)
                cidx[b][sl] = sbig[gsl] + cn
                didx[b][sl] = dbig[gsl]

        def issue(k, b):
            pltpu.async_copy(p_hbm.at[cidx[b]], prows[b], sg[b])
            pltpu.async_copy(
                q_hbm.at[c, pl.ds((qbase + k) * _CHUNK, _CHUNK)],
                qrows[b], sq[b])

        def wait_scatter(b):
            pltpu.make_async_copy(prows[b], acc.at[didx[b]], ss[b]).wait()

        def group(g, carry):
            g0 = g * _GRP
            pltpu.sync_copy(
                src_hbm.at[pl.ds(ibase + g0 * _CHUNK, _GRP * _CHUNK)], sbig)
            pltpu.sync_copy(
                dst_hbm.at[pl.ds(ibase + g0 * _CHUNK, _GRP * _CHUNK)], dbig)
            for j in range(_GRP):
                k = g0 + j
                b = j % 2

                if j == 0:
                    @pl.when(k >= 2)
                    def _():
                        wait_scatter(b)
                    build_idx(0, b)
                    issue(k, b)

                pltpu.make_async_copy(
                    p_hbm.at[cidx[b]], prows[b], sg[b]).wait()
                pltpu.make_async_copy(
                    q_hbm.at[c, pl.ds((qbase + k) * _CHUNK, _CHUNK)],
                    qrows[b], sq[b]).wait()

                if j < _GRP - 1:
                    @pl.when(k >= 1)
                    def _():
                        wait_scatter(1 - b)
                    build_idx(j + 1, 1 - b)
                    issue(k + 1, 1 - b)

                def row(jj, rc):
                    for cc in range(ncol):
                        sl = pl.ds(cc * _LANES, _LANES)
                        prows[b][jj, sl] = jnp.maximum(
                            prows[b][jj, sl] + qrows[b][jj, sl], 0.0)
                    return rc
                lax.fori_loop(0, _CHUNK, row, 0, unroll=2)

                pltpu.async_copy(prows[b], acc.at[didx[b]], ss[b], add=True)
            return carry
        lax.fori_loop(0, ngroups, group, 0)

        wait_scatter(0)
        wait_scatter(1)
        plsc.subcore_barrier()

        ob = cn + r0
        for k in range(nfull):
            pltpu.sync_copy(acc.at[pl.ds(r0 + k * _CHUNK, _CHUNK)],
                            out_hbm.at[pl.ds(ob + k * _CHUNK, _CHUNK)])
        if rem:
            pltpu.sync_copy(acc.at[pl.ds(r0 + nfull * _CHUNK, rem)],
                            out_hbm.at[pl.ds(ob + nfull * _CHUNK, rem)])

        @pl.when(s < nextra)
        def _():
            pltpu.sync_copy(acc.at[pl.ds(rz * _NS + s * 8, 8)],
                            out_hbm.at[pl.ds(cn + rz * _NS + s * 8, 8)])

    return sc_kernel(src1d, dst1d, p, q)


def kernel(h, edge_index, e, W1, b1, gamma, beta, run_mean, run_var, W2, b2):
    n, nd = h.shape
    ne, ed = e.shape
    emb = W1.shape[0]

    scale = gamma * lax.rsqrt(run_var + 1e-5)
    shift = beta - run_mean * scale
    w1s = W1 * scale[:, None]
    beff = b1 * scale + shift

    wa = jnp.zeros((nd, _W), jnp.float32).at[:, :emb].set(w1s[:, :nd].T)
    wb = jnp.zeros((ed, _W), jnp.float32).at[:, :emb].set(w1s[:, nd:].T)
    brow = jnp.zeros((1, _W), jnp.float32).at[0, :emb].set(beff)
    brow = brow.at[0, emb].set(1.0)
    w2e = jnp.zeros((_W, nd), jnp.float32).at[:emb, :].set(W2.T)
    w2e = w2e.at[emb, :].set(b2)
    wa_stack = jnp.stack([wa[:, :_HW], wa[:, _HW:]])
    wb_stack = jnp.stack([wb[:, :_HW], wb[:, _HW:]])
    br_stack = jnp.stack([brow[:, :_HW], brow[:, _HW:]])

    nchunks = ne // _CHUNK
    src1d = edge_index[0].astype(jnp.int32)
    dst1d = edge_index[1].astype(jnp.int32)

    p = _node_mm(h, wa_stack, n_block=1000)
    q, e_out = _edge_mm(e, wb_stack, br_stack, e_block=2000)
    s_stack = _sc_segment_sum(src1d, dst1d, p, q, n, nchunks)
    h_out = _out_mm(s_stack, w2e[:_HW], w2e[_HW:], n_block=1000)
    return (h_out, e_out)
